# Initial kernel scaffold; baseline (speedup 1.0000x reference)
import jax
import jax.numpy as jnp
from jax.experimental import pallas as pl


def kernel(x, edge_index, W_in, b_in, Wq, bq, Wk, bk, Wv, bv, Ws, bs, g1, be1, g2, be2, W1, b1, W2, b2, beta_p):
    raise NotImplementedError("write your pallas kernel here")


# speedup vs baseline: 13.1624x; 13.1624x over previous
import functools

import jax
import jax.numpy as jnp
from jax import lax
from jax.experimental import pallas as pl
from jax.experimental.pallas import tpu as pltpu
from jax.experimental.pallas import tpu_sc as plsc

N = 10000
E = 320000
H = 128
HEADS = 2
DH = H // HEADS
L = 6

NP = 10240
TRASH = N
NC, NS, LANES = 2, 16, 16
NW = NC * NS
EW = NP
EP = NW * EW
CH = 128
NCHUNK = EW // CH
CW = DH + 16
RB = 512
GRID = NP // RB


def _edge_body(q_hbm, k_hbm, v_hbm, src_hbm, dst_hbm, out_hbm,
               srcv, dstv, qrows, krows, vrows, orows, agg, semq, semk, semv):
  core = lax.axis_index("c")
  tid = lax.axis_index("s")
  zv = jnp.zeros((LANES,), jnp.float32)

  def zrow(i, _):
    for j in range(CW // LANES):
      orows[i, pl.ds(j * LANES, LANES)] = zv
    return 0
  lax.fori_loop(0, CH, zrow, 0)
  for cc in range(NP // NS // CH):
    pltpu.sync_copy(orows, agg.at[pl.ds(tid * (NP // NS) + cc * CH, CH)])
  plsc.subcore_barrier()

  base = (core * NS + tid) * EW
  lane = lax.iota(jnp.int32, LANES)

  def chunk(g, _):
    eb = base + g * CH
    pltpu.sync_copy(src_hbm.at[pl.ds(eb, CH)], srcv)
    pltpu.sync_copy(dst_hbm.at[pl.ds(eb, CH)], dstv)
    cq = pltpu.async_copy(q_hbm.at[dstv], qrows, semq)
    ck = pltpu.async_copy(k_hbm.at[srcv], krows, semk)
    cv = pltpu.async_copy(v_hbm.at[srcv], vrows, semv)
    cq.wait()
    ck.wait()
    cv.wait()

    def edge(e, _):
      acc = (qrows[e, pl.ds(0, LANES)] *
             krows[e, pl.ds(0, LANES)])
      for j in range(1, DH // LANES):
        acc = acc + (qrows[e, pl.ds(j * LANES, LANES)] *
                     krows[e, pl.ds(j * LANES, LANES)])
      d = jnp.sum(acc) * (1.0 / (DH ** 0.5))
      ev = jnp.exp(jnp.minimum(jnp.broadcast_to(d, (LANES,)), 60.0))
      for j in range(DH // LANES):
        orows[e, pl.ds(j * LANES, LANES)] = (
            vrows[e, pl.ds(j * LANES, LANES)] * ev)
      orows[e, pl.ds(DH, LANES)] = jnp.where(lane == 0, ev, 0.0)
      return 0

    lax.fori_loop(0, CH, edge, 0)
    pltpu.sync_copy(orows, agg.at[dstv], add=True)
    return 0

  lax.fori_loop(0, NCHUNK, chunk, 0)
  plsc.subcore_barrier()
  for cc in range(NP // NS // CH):
    r0 = tid * (NP // NS) + cc * CH
    pltpu.sync_copy(agg.at[pl.ds(r0, CH)],
                    out_hbm.at[core, pl.ds(r0, CH)])


_edge_kernel_cache = []


def _edge_kernel(q, k, v, src, dst):
  if not _edge_kernel_cache:
    mesh = plsc.VectorSubcoreMesh(
        core_axis_name="c", subcore_axis_name="s",
        num_cores=NC, num_subcores=NS)
    _edge_kernel_cache.append(pl.kernel(
        _edge_body,
        out_type=jax.ShapeDtypeStruct((NC, NP, CW), jnp.float32),
        mesh=mesh,
        compiler_params=pltpu.CompilerParams(
            needs_layout_passes=False, use_tc_tiling_on_sc=False),
        scratch_types=[
            pltpu.VMEM((CH,), jnp.int32),
            pltpu.VMEM((CH,), jnp.int32),
            pltpu.VMEM((CH, DH), jnp.float32),
            pltpu.VMEM((CH, DH), jnp.float32),
            pltpu.VMEM((CH, DH), jnp.float32),
            pltpu.VMEM((CH, CW), jnp.float32),
            pltpu.VMEM_SHARED((NP, CW), jnp.float32),
            pltpu.SemaphoreType.DMA,
            pltpu.SemaphoreType.DMA,
            pltpu.SemaphoreType.DMA,
        ],
    ))
  return _edge_kernel_cache[0](q, k, v, src, dst)


def _ln(x, g, b):
  m = jnp.mean(x, axis=-1, keepdims=True)
  v = jnp.mean((x - m) ** 2, axis=-1, keepdims=True)
  return (x - m) / jnp.sqrt(v + 1e-5) * g + b


def _in_body(x_ref, w_ref, b_ref, bp_ref, z_ref, acc_ref, betas_ref):
  z = jnp.maximum(
      jnp.dot(x_ref[...], w_ref[...], preferred_element_type=jnp.float32)
      + b_ref[...], 0.0)
  z_ref[...] = z
  e = jnp.exp(bp_ref[...])
  betas = e / jnp.sum(e)
  betas_ref[...] = betas
  acc_ref[...] = z * betas[0, 0]


def _input_kernel(xp, w, b, bp):
  return pl.pallas_call(
      _in_body,
      grid=(GRID,),
      in_specs=[
          pl.BlockSpec((RB, H), lambda i: (i, 0)),
          pl.BlockSpec((H, H), lambda i: (0, 0)),
          pl.BlockSpec((1, H), lambda i: (0, 0)),
          pl.BlockSpec((1, 8), lambda i: (0, 0)),
      ],
      out_specs=[
          pl.BlockSpec((RB, H), lambda i: (i, 0)),
          pl.BlockSpec((RB, H), lambda i: (i, 0)),
          pl.BlockSpec((1, 8), lambda i: (0, 0)),
      ],
      out_shape=[
          jax.ShapeDtypeStruct((NP, H), jnp.float32),
          jax.ShapeDtypeStruct((NP, H), jnp.float32),
          jax.ShapeDtypeStruct((1, 8), jnp.float32),
      ],
  )(xp, w, b, bp)


def _qkvs_body(z_ref, w_ref, b_ref, q0_ref, q1_ref, k0_ref, k1_ref,
               v0_ref, v1_ref, s_ref):
  y = jnp.dot(z_ref[...], w_ref[...], preferred_element_type=jnp.float32) \
      + b_ref[...]
  q0_ref[...] = y[:, 0 * DH:1 * DH]
  q1_ref[...] = y[:, 1 * DH:2 * DH]
  k0_ref[...] = y[:, 2 * DH:3 * DH]
  k1_ref[...] = y[:, 3 * DH:4 * DH]
  v0_ref[...] = y[:, 4 * DH:5 * DH]
  v1_ref[...] = y[:, 5 * DH:6 * DH]
  s_ref[...] = y[:, 3 * H:4 * H]


def _qkvs_kernel(z, wcat, bcat):
  return pl.pallas_call(
      _qkvs_body,
      grid=(GRID,),
      in_specs=[
          pl.BlockSpec((RB, H), lambda i: (i, 0)),
          pl.BlockSpec((H, 4 * H), lambda i: (0, 0)),
          pl.BlockSpec((1, 4 * H), lambda i: (0, 0)),
      ],
      out_specs=[pl.BlockSpec((RB, DH), lambda i: (i, 0))] * 6
      + [pl.BlockSpec((RB, H), lambda i: (i, 0))],
      out_shape=[jax.ShapeDtypeStruct((NP, DH), jnp.float32)] * 6
      + [jax.ShapeDtypeStruct((NP, H), jnp.float32)],
  )(z, wcat, bcat)


def _make_post_body(lidx):
  def body(z_ref, agg0_ref, agg1_ref, s_ref, g1_ref, be1_ref, w1_ref,
           b1_ref, w2_ref, b2_ref, g2_ref, be2_ref, betas_ref, accin_ref,
           zout_ref, accout_ref):
    a0 = agg0_ref[0] + agg0_ref[1]
    a1 = agg1_ref[0] + agg1_ref[1]
    att = jnp.concatenate(
        [a0[:, :DH] / (a0[:, DH:DH + 1] + 1e-16),
         a1[:, :DH] / (a1[:, DH:DH + 1] + 1e-16)], axis=1)
    att = att + s_ref[...]
    a = _ln(z_ref[...] + att, g1_ref[...], be1_ref[...])
    pre = jnp.dot(a, w1_ref[...], preferred_element_type=jnp.float32) \
        + b1_ref[...]
    hid = 0.5 * pre * (1.0 + lax.erf(pre * (2.0 ** -0.5)))
    f = jnp.dot(hid, w2_ref[...], preferred_element_type=jnp.float32) \
        + b2_ref[...]
    zl = jnp.maximum(_ln(a + f, g2_ref[...], be2_ref[...]), 0.0)
    zout_ref[...] = zl
    accout_ref[...] = accin_ref[...] + betas_ref[0, lidx + 1] * zl
  return body


def _post_kernel(lidx, z, aggs0, aggs1, s, g1, be1, w1, b1, w2, b2, g2,
                 be2, betas, acc):
  return pl.pallas_call(
      _make_post_body(lidx),
      grid=(GRID,),
      in_specs=[
          pl.BlockSpec((RB, H), lambda i: (i, 0)),
          pl.BlockSpec((NC, RB, CW), lambda i: (0, i, 0)),
          pl.BlockSpec((NC, RB, CW), lambda i: (0, i, 0)),
          pl.BlockSpec((RB, H), lambda i: (i, 0)),
          pl.BlockSpec((1, H), lambda i: (0, 0)),
          pl.BlockSpec((1, H), lambda i: (0, 0)),
          pl.BlockSpec((H, 2 * H), lambda i: (0, 0)),
          pl.BlockSpec((1, 2 * H), lambda i: (0, 0)),
          pl.BlockSpec((2 * H, H), lambda i: (0, 0)),
          pl.BlockSpec((1, H), lambda i: (0, 0)),
          pl.BlockSpec((1, H), lambda i: (0, 0)),
          pl.BlockSpec((1, H), lambda i: (0, 0)),
          pl.BlockSpec((1, 8), lambda i: (0, 0)),
          pl.BlockSpec((RB, H), lambda i: (i, 0)),
      ],
      out_specs=[
          pl.BlockSpec((RB, H), lambda i: (i, 0)),
          pl.BlockSpec((RB, H), lambda i: (i, 0)),
      ],
      out_shape=[
          jax.ShapeDtypeStruct((NP, H), jnp.float32),
          jax.ShapeDtypeStruct((NP, H), jnp.float32),
      ],
      input_output_aliases={13: 1},
  )(z, aggs0, aggs1, s, g1, be1, w1, b1, w2, b2, g2, be2, betas, acc)


def kernel(x, edge_index, W_in, b_in, Wq, bq, Wk, bk, Wv, bv, Ws, bs,
           g1, be1, g2, be2, W1, b1, W2, b2, beta_p):
  xp = jnp.pad(x, ((0, NP - N), (0, 0)))
  src = jnp.pad(edge_index[0], (0, EP - E))
  dst = jnp.pad(edge_index[1], (0, EP - E), constant_values=TRASH)
  bp = jnp.pad(beta_p, (0, 8 - (L + 1)), constant_values=-1e30)
  bp = bp.reshape(1, 8)

  z, acc, betas = _input_kernel(xp, W_in, b_in.reshape(1, H), bp)

  for l in range(L):
    wcat = jnp.concatenate([Wq[l], Wk[l], Wv[l], Ws[l]], axis=1)
    bcat = jnp.concatenate([bq[l], bk[l], bv[l], bs[l]]).reshape(1, 4 * H)
    q0, q1, k0, k1, v0, v1, s = _qkvs_kernel(z, wcat, bcat)
    aggs0 = _edge_kernel(q0, k0, v0, src, dst)
    aggs1 = _edge_kernel(q1, k1, v1, src, dst)
    z, acc = _post_kernel(
        l, z, aggs0, aggs1, s,
        g1[l].reshape(1, H), be1[l].reshape(1, H),
        W1[l], b1[l].reshape(1, 2 * H),
        W2[l], b2[l].reshape(1, H),
        g2[l].reshape(1, H), be2[l].reshape(1, H),
        betas, acc)

  return acc[:N]


# speedup vs baseline: 21.0485x; 1.5991x over previous
import functools

import jax
import jax.numpy as jnp
from jax import lax
from jax.experimental import pallas as pl
from jax.experimental.pallas import tpu as pltpu
from jax.experimental.pallas import tpu_sc as plsc

N = 10000
E = 320000
H = 128
HEADS = 2
DH = H // HEADS
L = 6

NP = 10240
TRASH = N
NC, NS, LANES = 2, 16, 16
NW = NC * NS
EW = NP
EP = NW * EW
CH = 128
NCHUNK = EW // CH
CW = DH + 16
RB = 512
GRID = NP // RB


def _edge_body(q_hbm, k_hbm, v_hbm, src_hbm, dst_hbm, out_hbm,
               srcb, dstb, q0, k0, v0, q1, k1, v1, orows, agg,
               sq0, sk0, sv0, sq1, sk1, sv1):
  core = lax.axis_index("c")
  tid = lax.axis_index("s")
  zv = jnp.zeros((LANES,), jnp.float32)

  def zrow(i, _):
    for j in range(CW // LANES):
      orows[i, pl.ds(j * LANES, LANES)] = zv
    return 0
  lax.fori_loop(0, CH, zrow, 0)
  for cc in range(NP // NS // CH):
    pltpu.sync_copy(orows, agg.at[pl.ds(tid * (NP // NS) + cc * CH, CH)])

  w = core * NS + tid
  pltpu.sync_copy(src_hbm.at[pl.ds(w * NCHUNK, NCHUNK)], srcb)
  pltpu.sync_copy(dst_hbm.at[pl.ds(w * NCHUNK, NCHUNK)], dstb)
  plsc.subcore_barrier()

  lane = lax.iota(jnp.int32, LANES)
  qb, kb, vb = (q0, q1), (k0, k1), (v0, v1)
  sqs, sks, svs = (sq0, sq1), (sk0, sk1), (sv0, sv1)

  def issue(g, p):
    pltpu.async_copy(q_hbm.at[dstb.at[g]], qb[p], sqs[p])
    pltpu.async_copy(k_hbm.at[srcb.at[g]], kb[p], sks[p])
    pltpu.async_copy(v_hbm.at[srcb.at[g]], vb[p], svs[p])

  def waitg(g, p):
    pltpu.make_async_copy(q_hbm.at[dstb.at[g]], qb[p], sqs[p]).wait()
    pltpu.make_async_copy(k_hbm.at[srcb.at[g]], kb[p], sks[p]).wait()
    pltpu.make_async_copy(v_hbm.at[srcb.at[g]], vb[p], svs[p]).wait()

  def compute(g, p):
    qr, kr, vr = qb[p], kb[p], vb[p]

    def edge(e, _):
      acc = qr[e, pl.ds(0, LANES)] * kr[e, pl.ds(0, LANES)]
      for j in range(1, DH // LANES):
        acc = acc + (qr[e, pl.ds(j * LANES, LANES)] *
                     kr[e, pl.ds(j * LANES, LANES)])
      d = jnp.sum(acc) * (1.0 / (DH ** 0.5))
      ev = jnp.exp(jnp.minimum(jnp.broadcast_to(d, (LANES,)), 60.0))
      for j in range(DH // LANES):
        orows[e, pl.ds(j * LANES, LANES)] = (
            vr[e, pl.ds(j * LANES, LANES)] * ev)
      orows[e, pl.ds(DH, LANES)] = jnp.where(lane == 0, ev, 0.0)
      return 0

    lax.fori_loop(0, CH, edge, 0)
    pltpu.sync_copy(orows, agg.at[dstb.at[g]], add=True)

  issue(0, 0)

  def outer(i, _):
    g = 2 * i
    issue(jnp.minimum(g + 1, NCHUNK - 1), 1)
    waitg(g, 0)
    compute(g, 0)
    issue(jnp.minimum(g + 2, NCHUNK - 1), 0)
    waitg(g + 1, 1)
    compute(g + 1, 1)
    return 0

  lax.fori_loop(0, NCHUNK // 2, outer, 0)
  waitg(NCHUNK - 1, 0)

  plsc.subcore_barrier()
  for cc in range(NP // NS // CH):
    r0 = tid * (NP // NS) + cc * CH
    pltpu.sync_copy(agg.at[pl.ds(r0, CH)],
                    out_hbm.at[core, pl.ds(r0, CH)])


_edge_kernel_cache = []


def _edge_kernel(q, k, v, src, dst):
  if not _edge_kernel_cache:
    mesh = plsc.VectorSubcoreMesh(
        core_axis_name="c", subcore_axis_name="s",
        num_cores=NC, num_subcores=NS)
    _edge_kernel_cache.append(pl.kernel(
        _edge_body,
        out_type=jax.ShapeDtypeStruct((NC, NP, CW), jnp.float32),
        mesh=mesh,
        compiler_params=pltpu.CompilerParams(
            needs_layout_passes=False, use_tc_tiling_on_sc=False),
        scratch_types=[
            pltpu.VMEM((NCHUNK, CH), jnp.int32),
            pltpu.VMEM((NCHUNK, CH), jnp.int32),
            pltpu.VMEM((CH, DH), jnp.float32),
            pltpu.VMEM((CH, DH), jnp.float32),
            pltpu.VMEM((CH, DH), jnp.float32),
            pltpu.VMEM((CH, DH), jnp.float32),
            pltpu.VMEM((CH, DH), jnp.float32),
            pltpu.VMEM((CH, DH), jnp.float32),
            pltpu.VMEM((CH, CW), jnp.float32),
            pltpu.VMEM_SHARED((NP, CW), jnp.float32),
        ] + [pltpu.SemaphoreType.DMA] * 6,
    ))
  return _edge_kernel_cache[0](q, k, v, src, dst)


def _ln(x, g, b):
  m = jnp.mean(x, axis=-1, keepdims=True)
  v = jnp.mean((x - m) ** 2, axis=-1, keepdims=True)
  return (x - m) / jnp.sqrt(v + 1e-5) * g + b


def _in_body(x_ref, w_ref, b_ref, bp_ref, z_ref, acc_ref, betas_ref):
  z = jnp.maximum(
      jnp.dot(x_ref[...], w_ref[...], preferred_element_type=jnp.float32)
      + b_ref[...], 0.0)
  z_ref[...] = z
  e = jnp.exp(bp_ref[...])
  betas = e / jnp.sum(e)
  betas_ref[...] = betas
  acc_ref[...] = z * betas[0, 0]


def _input_kernel(xp, w, b, bp):
  return pl.pallas_call(
      _in_body,
      grid=(GRID,),
      in_specs=[
          pl.BlockSpec((RB, H), lambda i: (i, 0)),
          pl.BlockSpec((H, H), lambda i: (0, 0)),
          pl.BlockSpec((1, H), lambda i: (0, 0)),
          pl.BlockSpec((1, 8), lambda i: (0, 0)),
      ],
      out_specs=[
          pl.BlockSpec((RB, H), lambda i: (i, 0)),
          pl.BlockSpec((RB, H), lambda i: (i, 0)),
          pl.BlockSpec((1, 8), lambda i: (0, 0)),
      ],
      out_shape=[
          jax.ShapeDtypeStruct((NP, H), jnp.float32),
          jax.ShapeDtypeStruct((NP, H), jnp.float32),
          jax.ShapeDtypeStruct((1, 8), jnp.float32),
      ],
  )(xp, w, b, bp)


def _qkvs_body(z_ref, w_ref, b_ref, q0_ref, q1_ref, k0_ref, k1_ref,
               v0_ref, v1_ref, s_ref):
  y = jnp.dot(z_ref[...], w_ref[...], preferred_element_type=jnp.float32) \
      + b_ref[...]
  q0_ref[...] = y[:, 0 * DH:1 * DH]
  q1_ref[...] = y[:, 1 * DH:2 * DH]
  k0_ref[...] = y[:, 2 * DH:3 * DH]
  k1_ref[...] = y[:, 3 * DH:4 * DH]
  v0_ref[...] = y[:, 4 * DH:5 * DH]
  v1_ref[...] = y[:, 5 * DH:6 * DH]
  s_ref[...] = y[:, 3 * H:4 * H]


def _qkvs_kernel(z, wcat, bcat):
  return pl.pallas_call(
      _qkvs_body,
      grid=(GRID,),
      in_specs=[
          pl.BlockSpec((RB, H), lambda i: (i, 0)),
          pl.BlockSpec((H, 4 * H), lambda i: (0, 0)),
          pl.BlockSpec((1, 4 * H), lambda i: (0, 0)),
      ],
      out_specs=[pl.BlockSpec((RB, DH), lambda i: (i, 0))] * 6
      + [pl.BlockSpec((RB, H), lambda i: (i, 0))],
      out_shape=[jax.ShapeDtypeStruct((NP, DH), jnp.float32)] * 6
      + [jax.ShapeDtypeStruct((NP, H), jnp.float32)],
  )(z, wcat, bcat)


def _make_post_body(lidx):
  def body(z_ref, agg0_ref, agg1_ref, s_ref, g1_ref, be1_ref, w1_ref,
           b1_ref, w2_ref, b2_ref, g2_ref, be2_ref, betas_ref, accin_ref,
           zout_ref, accout_ref):
    a0 = agg0_ref[0] + agg0_ref[1]
    a1 = agg1_ref[0] + agg1_ref[1]
    att = jnp.concatenate(
        [a0[:, :DH] / (a0[:, DH:DH + 1] + 1e-16),
         a1[:, :DH] / (a1[:, DH:DH + 1] + 1e-16)], axis=1)
    att = att + s_ref[...]
    a = _ln(z_ref[...] + att, g1_ref[...], be1_ref[...])
    pre = jnp.dot(a, w1_ref[...], preferred_element_type=jnp.float32) \
        + b1_ref[...]
    hid = 0.5 * pre * (1.0 + lax.erf(pre * (2.0 ** -0.5)))
    f = jnp.dot(hid, w2_ref[...], preferred_element_type=jnp.float32) \
        + b2_ref[...]
    zl = jnp.maximum(_ln(a + f, g2_ref[...], be2_ref[...]), 0.0)
    zout_ref[...] = zl
    accout_ref[...] = accin_ref[...] + betas_ref[0, lidx + 1] * zl
  return body


def _post_kernel(lidx, z, aggs0, aggs1, s, g1, be1, w1, b1, w2, b2, g2,
                 be2, betas, acc):
  return pl.pallas_call(
      _make_post_body(lidx),
      grid=(GRID,),
      in_specs=[
          pl.BlockSpec((RB, H), lambda i: (i, 0)),
          pl.BlockSpec((NC, RB, CW), lambda i: (0, i, 0)),
          pl.BlockSpec((NC, RB, CW), lambda i: (0, i, 0)),
          pl.BlockSpec((RB, H), lambda i: (i, 0)),
          pl.BlockSpec((1, H), lambda i: (0, 0)),
          pl.BlockSpec((1, H), lambda i: (0, 0)),
          pl.BlockSpec((H, 2 * H), lambda i: (0, 0)),
          pl.BlockSpec((1, 2 * H), lambda i: (0, 0)),
          pl.BlockSpec((2 * H, H), lambda i: (0, 0)),
          pl.BlockSpec((1, H), lambda i: (0, 0)),
          pl.BlockSpec((1, H), lambda i: (0, 0)),
          pl.BlockSpec((1, H), lambda i: (0, 0)),
          pl.BlockSpec((1, 8), lambda i: (0, 0)),
          pl.BlockSpec((RB, H), lambda i: (i, 0)),
      ],
      out_specs=[
          pl.BlockSpec((RB, H), lambda i: (i, 0)),
          pl.BlockSpec((RB, H), lambda i: (i, 0)),
      ],
      out_shape=[
          jax.ShapeDtypeStruct((NP, H), jnp.float32),
          jax.ShapeDtypeStruct((NP, H), jnp.float32),
      ],
      input_output_aliases={13: 1},
  )(z, aggs0, aggs1, s, g1, be1, w1, b1, w2, b2, g2, be2, betas, acc)


def kernel(x, edge_index, W_in, b_in, Wq, bq, Wk, bk, Wv, bv, Ws, bs,
           g1, be1, g2, be2, W1, b1, W2, b2, beta_p):
  xp = jnp.pad(x, ((0, NP - N), (0, 0)))
  src = jnp.pad(edge_index[0], (0, EP - E)).reshape(EP // CH, CH)
  dst = jnp.pad(edge_index[1], (0, EP - E),
                constant_values=TRASH).reshape(EP // CH, CH)
  bp = jnp.pad(beta_p, (0, 8 - (L + 1)), constant_values=-1e30)
  bp = bp.reshape(1, 8)

  z, acc, betas = _input_kernel(xp, W_in, b_in.reshape(1, H), bp)

  for l in range(L):
    wcat = jnp.concatenate([Wq[l], Wk[l], Wv[l], Ws[l]], axis=1)
    bcat = jnp.concatenate([bq[l], bk[l], bv[l], bs[l]]).reshape(1, 4 * H)
    q0, q1, k0, k1, v0, v1, s = _qkvs_kernel(z, wcat, bcat)
    aggs0 = _edge_kernel(q0, k0, v0, src, dst)
    aggs1 = _edge_kernel(q1, k1, v1, src, dst)
    z, acc = _post_kernel(
        l, z, aggs0, aggs1, s,
        g1[l].reshape(1, H), be1[l].reshape(1, H),
        W1[l], b1[l].reshape(1, 2 * H),
        W2[l], b2[l].reshape(1, H),
        g2[l].reshape(1, H), be2[l].reshape(1, H),
        betas, acc)

  return acc[:N]


# speedup vs baseline: 32.1705x; 1.5284x over previous
import functools

import jax
import jax.numpy as jnp
from jax import lax
from jax.experimental import pallas as pl
from jax.experimental.pallas import tpu as pltpu
from jax.experimental.pallas import tpu_sc as plsc

N = 10000
E = 320000
H = 128
HEADS = 2
DH = H // HEADS
L = 6

NP = 10240
TRASH = N
NC, NS, LANES = 2, 16, 16
NW = NC * NS
EW = NP
EP = NW * EW
CH = 128
NCHUNK = EW // CH
CW = DH + 16
RB = 512
GRID = NP // RB


def _edge_body(q_hbm, k_hbm, v_hbm, src_hbm, dst_hbm, out_hbm,
               srcb, dstb, q0, k0, v0, q1, k1, v1, o0, agg,
               sq0, sk0, sv0, sq1, sk1, sv1):
  core = lax.axis_index("c")
  tid = lax.axis_index("s")
  zv = jnp.zeros((LANES,), jnp.float32)

  def zrow(i, _):
    for j in range(CW // LANES):
      o0[i, pl.ds(j * LANES, LANES)] = zv
    return 0
  lax.fori_loop(0, CH, zrow, 0)
  for cc in range(NP // NS // CH):
    pltpu.sync_copy(o0, agg.at[pl.ds(tid * (NP // NS) + cc * CH, CH)])

  w = core * NS + tid
  pltpu.sync_copy(src_hbm.at[pl.ds(w * NCHUNK, NCHUNK)], srcb)
  pltpu.sync_copy(dst_hbm.at[pl.ds(w * NCHUNK, NCHUNK)], dstb)
  plsc.subcore_barrier()

  lane = lax.iota(jnp.int32, LANES)
  qb, kb, vb, ob = (q0, q1), (k0, k1), (v0, v1), (o0, o0)
  sqs, sks, svs = (sq0, sq1), (sk0, sk1), (sv0, sv1)

  def issue(g, p):
    pltpu.async_copy(q_hbm.at[dstb.at[g]], qb[p], sqs[p])
    pltpu.async_copy(k_hbm.at[srcb.at[g]], kb[p], sks[p])
    pltpu.async_copy(v_hbm.at[srcb.at[g]], vb[p], svs[p])

  def waitg(g, p):
    pltpu.make_async_copy(q_hbm.at[dstb.at[g]], qb[p], sqs[p]).wait()
    pltpu.make_async_copy(k_hbm.at[srcb.at[g]], kb[p], sks[p]).wait()
    pltpu.make_async_copy(v_hbm.at[srcb.at[g]], vb[p], svs[p]).wait()

  def compute(g, p):
    qr, kr, vr, orow = qb[p], kb[p], vb[p], ob[p]

    @plsc.parallel_loop(0, CH, step=1, unroll=4)
    def edge(e):
      acc = qr[e, pl.ds(0, LANES)] * kr[e, pl.ds(0, LANES)]
      for j in range(1, DH // LANES):
        acc = acc + (qr[e, pl.ds(j * LANES, LANES)] *
                     kr[e, pl.ds(j * LANES, LANES)])
      d = jnp.sum(acc) * (1.0 / (DH ** 0.5))
      ev = jnp.exp(jnp.minimum(jnp.broadcast_to(d, (LANES,)), 60.0))
      for j in range(DH // LANES):
        orow[e, pl.ds(j * LANES, LANES)] = (
            vr[e, pl.ds(j * LANES, LANES)] * ev)
      orow[e, pl.ds(DH, LANES)] = jnp.where(lane == 0, ev, 0.0)

    pltpu.sync_copy(orow, agg.at[dstb.at[g]], add=True)

  issue(0, 0)

  def outer(i, _):
    g = 2 * i
    issue(jnp.minimum(g + 1, NCHUNK - 1), 1)
    waitg(g, 0)
    compute(g, 0)
    issue(jnp.minimum(g + 2, NCHUNK - 1), 0)
    waitg(g + 1, 1)
    compute(g + 1, 1)
    return 0

  lax.fori_loop(0, NCHUNK // 2, outer, 0)
  waitg(NCHUNK - 1, 0)

  plsc.subcore_barrier()
  for cc in range(NP // NS // CH):
    r0 = tid * (NP // NS) + cc * CH
    pltpu.sync_copy(agg.at[pl.ds(r0, CH)],
                    out_hbm.at[core, pl.ds(r0, CH)])


_edge_kernel_cache = []


def _edge_kernel(q, k, v, src, dst):
  if not _edge_kernel_cache:
    mesh = plsc.VectorSubcoreMesh(
        core_axis_name="c", subcore_axis_name="s",
        num_cores=NC, num_subcores=NS)
    _edge_kernel_cache.append(pl.kernel(
        _edge_body,
        out_type=jax.ShapeDtypeStruct((NC, NP, CW), jnp.float32),
        mesh=mesh,
        compiler_params=pltpu.CompilerParams(
            needs_layout_passes=False, use_tc_tiling_on_sc=False),
        scratch_types=[
            pltpu.VMEM((NCHUNK, CH), jnp.int32),
            pltpu.VMEM((NCHUNK, CH), jnp.int32),
            pltpu.VMEM((CH, DH), jnp.float32),
            pltpu.VMEM((CH, DH), jnp.float32),
            pltpu.VMEM((CH, DH), jnp.float32),
            pltpu.VMEM((CH, DH), jnp.float32),
            pltpu.VMEM((CH, DH), jnp.float32),
            pltpu.VMEM((CH, DH), jnp.float32),
            pltpu.VMEM((CH, CW), jnp.float32),
            pltpu.VMEM_SHARED((NP, CW), jnp.float32),
        ] + [pltpu.SemaphoreType.DMA] * 6,
    ))
  return _edge_kernel_cache[0](q, k, v, src, dst)


def _ln(x, g, b):
  m = jnp.mean(x, axis=-1, keepdims=True)
  v = jnp.mean((x - m) ** 2, axis=-1, keepdims=True)
  return (x - m) / jnp.sqrt(v + 1e-5) * g + b


def _in_body(x_ref, w_ref, b_ref, bp_ref, z_ref, acc_ref, betas_ref):
  z = jnp.maximum(
      jnp.dot(x_ref[...], w_ref[...], preferred_element_type=jnp.float32)
      + b_ref[...], 0.0)
  z_ref[...] = z
  e = jnp.exp(bp_ref[...])
  betas = e / jnp.sum(e)
  betas_ref[...] = betas
  acc_ref[...] = z * betas[0, 0]


def _input_kernel(xp, w, b, bp):
  return pl.pallas_call(
      _in_body,
      grid=(GRID,),
      in_specs=[
          pl.BlockSpec((RB, H), lambda i: (i, 0)),
          pl.BlockSpec((H, H), lambda i: (0, 0)),
          pl.BlockSpec((1, H), lambda i: (0, 0)),
          pl.BlockSpec((1, 8), lambda i: (0, 0)),
      ],
      out_specs=[
          pl.BlockSpec((RB, H), lambda i: (i, 0)),
          pl.BlockSpec((RB, H), lambda i: (i, 0)),
          pl.BlockSpec((1, 8), lambda i: (0, 0)),
      ],
      out_shape=[
          jax.ShapeDtypeStruct((NP, H), jnp.float32),
          jax.ShapeDtypeStruct((NP, H), jnp.float32),
          jax.ShapeDtypeStruct((1, 8), jnp.float32),
      ],
  )(xp, w, b, bp)


def _qkvs_body(z_ref, w_ref, b_ref, q0_ref, q1_ref, k0_ref, k1_ref,
               v0_ref, v1_ref, s_ref):
  y = jnp.dot(z_ref[...], w_ref[...], preferred_element_type=jnp.float32) \
      + b_ref[...]
  q0_ref[...] = y[:, 0 * DH:1 * DH]
  q1_ref[...] = y[:, 1 * DH:2 * DH]
  k0_ref[...] = y[:, 2 * DH:3 * DH]
  k1_ref[...] = y[:, 3 * DH:4 * DH]
  v0_ref[...] = y[:, 4 * DH:5 * DH]
  v1_ref[...] = y[:, 5 * DH:6 * DH]
  s_ref[...] = y[:, 3 * H:4 * H]


def _qkvs_kernel(z, wcat, bcat):
  return pl.pallas_call(
      _qkvs_body,
      grid=(GRID,),
      in_specs=[
          pl.BlockSpec((RB, H), lambda i: (i, 0)),
          pl.BlockSpec((H, 4 * H), lambda i: (0, 0)),
          pl.BlockSpec((1, 4 * H), lambda i: (0, 0)),
      ],
      out_specs=[pl.BlockSpec((RB, DH), lambda i: (i, 0))] * 6
      + [pl.BlockSpec((RB, H), lambda i: (i, 0))],
      out_shape=[jax.ShapeDtypeStruct((NP, DH), jnp.float32)] * 6
      + [jax.ShapeDtypeStruct((NP, H), jnp.float32)],
  )(z, wcat, bcat)


def _make_post_body(lidx):
  def body(z_ref, agg0_ref, agg1_ref, s_ref, g1_ref, be1_ref, w1_ref,
           b1_ref, w2_ref, b2_ref, g2_ref, be2_ref, betas_ref, accin_ref,
           zout_ref, accout_ref):
    a0 = agg0_ref[0] + agg0_ref[1]
    a1 = agg1_ref[0] + agg1_ref[1]
    att = jnp.concatenate(
        [a0[:, :DH] / (a0[:, DH:DH + 1] + 1e-16),
         a1[:, :DH] / (a1[:, DH:DH + 1] + 1e-16)], axis=1)
    att = att + s_ref[...]
    a = _ln(z_ref[...] + att, g1_ref[...], be1_ref[...])
    pre = jnp.dot(a, w1_ref[...], preferred_element_type=jnp.float32) \
        + b1_ref[...]
    hid = 0.5 * pre * (1.0 + lax.erf(pre * (2.0 ** -0.5)))
    f = jnp.dot(hid, w2_ref[...], preferred_element_type=jnp.float32) \
        + b2_ref[...]
    zl = jnp.maximum(_ln(a + f, g2_ref[...], be2_ref[...]), 0.0)
    zout_ref[...] = zl
    accout_ref[...] = accin_ref[...] + betas_ref[0, lidx + 1] * zl
  return body


def _post_kernel(lidx, z, aggs0, aggs1, s, g1, be1, w1, b1, w2, b2, g2,
                 be2, betas, acc):
  return pl.pallas_call(
      _make_post_body(lidx),
      grid=(GRID,),
      in_specs=[
          pl.BlockSpec((RB, H), lambda i: (i, 0)),
          pl.BlockSpec((NC, RB, CW), lambda i: (0, i, 0)),
          pl.BlockSpec((NC, RB, CW), lambda i: (0, i, 0)),
          pl.BlockSpec((RB, H), lambda i: (i, 0)),
          pl.BlockSpec((1, H), lambda i: (0, 0)),
          pl.BlockSpec((1, H), lambda i: (0, 0)),
          pl.BlockSpec((H, 2 * H), lambda i: (0, 0)),
          pl.BlockSpec((1, 2 * H), lambda i: (0, 0)),
          pl.BlockSpec((2 * H, H), lambda i: (0, 0)),
          pl.BlockSpec((1, H), lambda i: (0, 0)),
          pl.BlockSpec((1, H), lambda i: (0, 0)),
          pl.BlockSpec((1, H), lambda i: (0, 0)),
          pl.BlockSpec((1, 8), lambda i: (0, 0)),
          pl.BlockSpec((RB, H), lambda i: (i, 0)),
      ],
      out_specs=[
          pl.BlockSpec((RB, H), lambda i: (i, 0)),
          pl.BlockSpec((RB, H), lambda i: (i, 0)),
      ],
      out_shape=[
          jax.ShapeDtypeStruct((NP, H), jnp.float32),
          jax.ShapeDtypeStruct((NP, H), jnp.float32),
      ],
      input_output_aliases={13: 1},
  )(z, aggs0, aggs1, s, g1, be1, w1, b1, w2, b2, g2, be2, betas, acc)


def kernel(x, edge_index, W_in, b_in, Wq, bq, Wk, bk, Wv, bv, Ws, bs,
           g1, be1, g2, be2, W1, b1, W2, b2, beta_p):
  xp = jnp.pad(x, ((0, NP - N), (0, 0)))
  src = jnp.pad(edge_index[0], (0, EP - E)).reshape(EP // CH, CH)
  dst = jnp.pad(edge_index[1], (0, EP - E),
                constant_values=TRASH).reshape(EP // CH, CH)
  bp = jnp.pad(beta_p, (0, 8 - (L + 1)), constant_values=-1e30)
  bp = bp.reshape(1, 8)

  z, acc, betas = _input_kernel(xp, W_in, b_in.reshape(1, H), bp)

  for l in range(L):
    wcat = jnp.concatenate([Wq[l], Wk[l], Wv[l], Ws[l]], axis=1)
    bcat = jnp.concatenate([bq[l], bk[l], bv[l], bs[l]]).reshape(1, 4 * H)
    q0, q1, k0, k1, v0, v1, s = _qkvs_kernel(z, wcat, bcat)
    aggs0 = _edge_kernel(q0, k0, v0, src, dst)
    aggs1 = _edge_kernel(q1, k1, v1, src, dst)
    z, acc = _post_kernel(
        l, z, aggs0, aggs1, s,
        g1[l].reshape(1, H), be1[l].reshape(1, H),
        W1[l], b1[l].reshape(1, 2 * H),
        W2[l], b2[l].reshape(1, H),
        g2[l].reshape(1, H), be2[l].reshape(1, H),
        betas, acc)

  return acc[:N]
